# baseline (speedup 1.0000x reference)
import functools
import math

import jax
import jax.numpy as jnp
from jax.experimental import pallas as pl

_SIZE = 100000
_PAD = 0
_SMOOTHING = 0.1
_CONF = 1.0 - _SMOOTHING
_EPS = _SMOOTHING / (_SIZE - 2)
_C1 = _SMOOTHING * math.log(_EPS) + _CONF * math.log(_CONF)

_RB = 8
_K = 4


def _loss_kernel(*refs):
    x_refs = refs[:_K]
    tgt_ref, out_ref = refs[_K], refs[_K + 1]
    i = pl.program_id(0)

    @pl.when(i == 0)
    def _init():
        out_ref[...] = jnp.zeros((1, 1), jnp.float32)

    acc = jnp.zeros((1, 1), jnp.float32)
    for k in range(_K):
        xb = x_refs[k][...]
        tgt = tgt_ref[k * _RB:(k + 1) * _RB, :]
        col = jax.lax.broadcasted_iota(jnp.int32, xb.shape, 1)
        w = jnp.where(col == tgt, -_CONF, -_EPS)
        ws = jnp.sum(w * xb, axis=1, keepdims=True)
        x0 = xb[:, 0:1]
        valid = (tgt != _PAD).astype(jnp.float32)
        per_row = _C1 + ws + _EPS * x0
        acc += jnp.sum(valid * per_row, axis=0, keepdims=True)
    out_ref[...] += acc


@functools.partial(jax.jit, static_argnames=())
def kernel(x, target):
    n = x.shape[0]
    tgt = target.astype(jnp.int32).reshape(n, 1)
    x_specs = [
        pl.BlockSpec((_RB, _SIZE), functools.partial(lambda k, i: (i * _K + k, 0), k))
        for k in range(_K)
    ]
    out = pl.pallas_call(
        _loss_kernel,
        grid=(n // (_K * _RB),),
        in_specs=x_specs + [pl.BlockSpec((_K * _RB, 1), lambda i: (i, 0))],
        out_specs=pl.BlockSpec((1, 1), lambda i: (0, 0)),
        out_shape=jax.ShapeDtypeStruct((1, 1), jnp.float32),
    )(*([x] * _K), tgt)
    return out[0, 0]


# speedup vs baseline: 3.4089x; 3.4089x over previous
import functools
import math

import jax
import jax.numpy as jnp
from jax.experimental import pallas as pl

_SIZE = 100000
_PAD = 0
_SMOOTHING = 0.1
_CONF = 1.0 - _SMOOTHING
_EPS = _SMOOTHING / (_SIZE - 2)
_C1 = _SMOOTHING * math.log(_EPS) + _CONF * math.log(_CONF)

_VB = 2000


def _loss_kernel(xt_ref, tgt_ref, out_ref):
    i = pl.program_id(0)

    @pl.when(i == 0)
    def _init():
        out_ref[...] = jnp.zeros((1, 1), jnp.float32)

    xb = xt_ref[...]
    tgt = tgt_ref[...]
    valid = tgt != _PAD
    jrow = jax.lax.broadcasted_iota(jnp.int32, xb.shape, 0)
    tloc = tgt - i * _VB
    w = jnp.where(jrow == tloc, -_CONF, -_EPS)
    s_cols = jnp.sum(w * xb, axis=0, keepdims=True)
    acc = jnp.sum(jnp.where(valid, s_cols, 0.0), axis=1, keepdims=True)

    @pl.when(i == 0)
    def _pad_col_and_const():
        x0 = xt_ref[0:1, :]
        extra = jnp.where(valid, _EPS * x0 + _C1, 0.0)
        out_ref[...] += jnp.sum(extra, axis=1, keepdims=True)

    out_ref[...] += acc


@functools.partial(jax.jit, static_argnames=())
def kernel(x, target):
    n = x.shape[0]
    xt = x.T
    tgt = target.astype(jnp.int32).reshape(1, n)
    out = pl.pallas_call(
        _loss_kernel,
        grid=(_SIZE // _VB,),
        in_specs=[
            pl.BlockSpec((_VB, n), lambda i: (i, 0)),
            pl.BlockSpec((1, n), lambda i: (0, 0)),
        ],
        out_specs=pl.BlockSpec((1, 1), lambda i: (0, 0)),
        out_shape=jax.ShapeDtypeStruct((1, 1), jnp.float32),
    )(xt, tgt)
    return out[0, 0]


# speedup vs baseline: 3.7583x; 1.1025x over previous
import functools
import math

import jax
import jax.numpy as jnp
from jax.experimental import pallas as pl

_SIZE = 100000
_PAD = 0
_SMOOTHING = 0.1
_CONF = 1.0 - _SMOOTHING
_EPS = _SMOOTHING / (_SIZE - 2)
_C1 = _SMOOTHING * math.log(_EPS) + _CONF * math.log(_CONF)

_VB = 4000


def _loss_kernel(xt_ref, tgt_ref, out_ref):
    i = pl.program_id(0)

    @pl.when(i == 0)
    def _init():
        out_ref[...] = jnp.zeros((1, 1), jnp.float32)

    xb = xt_ref[...]
    tgt = tgt_ref[...]
    valid = tgt != _PAD
    jrow = jax.lax.broadcasted_iota(jnp.int32, xb.shape, 0)
    tloc = tgt - i * _VB
    w = jnp.where(jrow == tloc, -_CONF, -_EPS)
    s_cols = jnp.sum(w * xb, axis=0, keepdims=True)
    acc = jnp.sum(jnp.where(valid, s_cols, 0.0), axis=1, keepdims=True)

    @pl.when(i == 0)
    def _pad_col_and_const():
        x0 = xt_ref[0:1, :]
        extra = jnp.where(valid, _EPS * x0 + _C1, 0.0)
        out_ref[...] += jnp.sum(extra, axis=1, keepdims=True)

    out_ref[...] += acc


@functools.partial(jax.jit, static_argnames=())
def kernel(x, target):
    n = x.shape[0]
    xt = x.T
    tgt = target.astype(jnp.int32).reshape(1, n)
    out = pl.pallas_call(
        _loss_kernel,
        grid=(_SIZE // _VB,),
        in_specs=[
            pl.BlockSpec((_VB, n), lambda i: (i, 0)),
            pl.BlockSpec((1, n), lambda i: (0, 0)),
        ],
        out_specs=pl.BlockSpec((1, 1), lambda i: (0, 0)),
        out_shape=jax.ShapeDtypeStruct((1, 1), jnp.float32),
    )(xt, tgt)
    return out[0, 0]


# speedup vs baseline: 3.8065x; 1.0128x over previous
import functools
import math

import jax
import jax.numpy as jnp
from jax.experimental import pallas as pl

_SIZE = 100000
_PAD = 0
_SMOOTHING = 0.1
_CONF = 1.0 - _SMOOTHING
_EPS = _SMOOTHING / (_SIZE - 2)
_C1 = _SMOOTHING * math.log(_EPS) + _CONF * math.log(_CONF)

_VB = 5000


def _loss_kernel(xt_ref, tgt_ref, out_ref):
    i = pl.program_id(0)

    @pl.when(i == 0)
    def _init():
        out_ref[...] = jnp.zeros((1, 1), jnp.float32)

    xb = xt_ref[...]
    tgt = tgt_ref[...]
    valid = tgt != _PAD
    jrow = jax.lax.broadcasted_iota(jnp.int32, xb.shape, 0)
    tloc = tgt - i * _VB
    w = jnp.where(jrow == tloc, -_CONF, -_EPS)
    s_cols = jnp.sum(w * xb, axis=0, keepdims=True)
    acc = jnp.sum(jnp.where(valid, s_cols, 0.0), axis=1, keepdims=True)

    @pl.when(i == 0)
    def _pad_col_and_const():
        x0 = xt_ref[0:1, :]
        extra = jnp.where(valid, _EPS * x0 + _C1, 0.0)
        out_ref[...] += jnp.sum(extra, axis=1, keepdims=True)

    out_ref[...] += acc


@functools.partial(jax.jit, static_argnames=())
def kernel(x, target):
    n = x.shape[0]
    xt = x.T
    tgt = target.astype(jnp.int32).reshape(1, n)
    out = pl.pallas_call(
        _loss_kernel,
        grid=(_SIZE // _VB,),
        in_specs=[
            pl.BlockSpec((_VB, n), lambda i: (i, 0)),
            pl.BlockSpec((1, n), lambda i: (0, 0)),
        ],
        out_specs=pl.BlockSpec((1, 1), lambda i: (0, 0)),
        out_shape=jax.ShapeDtypeStruct((1, 1), jnp.float32),
    )(xt, tgt)
    return out[0, 0]
